# baseline (speedup 1.0000x reference)
import jax
import jax.numpy as jnp
from jax.experimental import pallas as pl
from jax.experimental.pallas import tpu as pltpu

_K = 3
_G = 8


def _merge3(a1, a2, a3, b1, b2, b3):
    c1 = jnp.maximum(a1, b1)
    m1 = jnp.minimum(a1, b1)
    hi2 = jnp.maximum(a2, b2)
    lo2 = jnp.minimum(a2, b2)
    hi3 = jnp.maximum(a3, b3)
    c2 = jnp.maximum(m1, hi2)
    c3 = jnp.maximum(jnp.minimum(m1, hi2), jnp.maximum(lo2, hi3))
    return c1, c2, c3


def _fused_kernel(q_ref, s_ref, o_ref):
    _, QC, P, D = q_ref.shape
    _, W, SP, _ = s_ref.shape
    R = QC * P

    qm = q_ref[0].reshape(R, D)
    xs = jax.lax.dot_general(
        s_ref[0].reshape(W * SP, D), qm,
        dimension_numbers=(((1,), (1,)), ((), ())),
        preferred_element_type=jnp.float32,
    )
    per_way = []
    for w in range(W):
        x = xs[w * SP:(w + 1) * SP]
        t1 = x[0:_G]
        v = x[_G:2 * _G]
        t2 = jnp.minimum(t1, v)
        t1 = jnp.maximum(t1, v)
        v = x[2 * _G:3 * _G]
        lo1 = jnp.minimum(t1, v)
        t1 = jnp.maximum(t1, v)
        t3 = jnp.minimum(t2, lo1)
        t2 = jnp.maximum(t2, lo1)
        for i in range(3, SP // _G):
            v = x[i * _G:(i + 1) * _G]
            lo1 = jnp.minimum(t1, v)
            t1 = jnp.maximum(t1, v)
            lo2 = jnp.minimum(t2, lo1)
            t2 = jnp.maximum(t2, lo1)
            t3 = jnp.maximum(t3, lo2)
        for s in (4, 2, 1):
            b1 = pltpu.roll(t1, s, 0)
            b2 = pltpu.roll(t2, s, 0)
            b3 = pltpu.roll(t3, s, 0)
            t1, t2, t3 = _merge3(t1, t2, t3, b1, b2, b3)
        tsum = (t1 + t2 + t3)[0:1]
        per_q = tsum.reshape(QC, P).sum(axis=-1) * (1.0 / (P * _K))
        per_way.append(per_q)
    o_ref[0, 0] = jnp.stack(per_way, axis=-1)


def kernel(query_fea, support_fea):
    b, q, p, d = query_fea.shape
    _, way, shot, _, _ = support_fea.shape
    support = support_fea.reshape(b, way, shot * p, d)

    qc = q
    for cand in (30, 15, 10, 6, 5, 3, 2, 1):
        if q % cand == 0:
            qc = cand
            break

    out = pl.pallas_call(
        _fused_kernel,
        grid=(b, q // qc),
        in_specs=[
            pl.BlockSpec((1, qc, p, d), lambda i, j: (i, j, 0, 0)),
            pl.BlockSpec((1, way, shot * p, d), lambda i, j: (i, 0, 0, 0)),
        ],
        out_specs=pl.BlockSpec((1, 1, qc, way), lambda i, j: (i, j, 0, 0)),
        out_shape=jax.ShapeDtypeStruct((b, q // qc, qc, way), jnp.float32),
        compiler_params=pltpu.CompilerParams(
            dimension_semantics=("parallel", "parallel"),
        ),
    )(query_fea, support)
    return out.reshape(b, q, way)


# speedup vs baseline: 1.0255x; 1.0255x over previous
import jax
import jax.numpy as jnp
from jax.experimental import pallas as pl
from jax.experimental.pallas import tpu as pltpu

_K = 3
_G = 8


def _merge3(a1, a2, a3, b1, b2, b3):
    c1 = jnp.maximum(a1, b1)
    m1 = jnp.minimum(a1, b1)
    hi2 = jnp.maximum(a2, b2)
    lo2 = jnp.minimum(a2, b2)
    hi3 = jnp.maximum(a3, b3)
    c2 = jnp.maximum(m1, hi2)
    c3 = jnp.maximum(jnp.minimum(m1, hi2), jnp.maximum(lo2, hi3))
    return c1, c2, c3


def _fused_kernel(q_ref, s_ref, o_ref):
    _, QC, P, D = q_ref.shape
    _, W, SP, _ = s_ref.shape
    R = QC * P

    qm = q_ref[0].reshape(R, D)
    per_way = []
    for w in range(W):
        x = jax.lax.dot_general(
            s_ref[0, w], qm,
            dimension_numbers=(((1,), (1,)), ((), ())),
            preferred_element_type=jnp.float32,
        )
        t1 = x[0:_G]
        v = x[_G:2 * _G]
        t2 = jnp.minimum(t1, v)
        t1 = jnp.maximum(t1, v)
        v = x[2 * _G:3 * _G]
        lo1 = jnp.minimum(t1, v)
        t1 = jnp.maximum(t1, v)
        t3 = jnp.minimum(t2, lo1)
        t2 = jnp.maximum(t2, lo1)
        for i in range(3, SP // _G):
            v = x[i * _G:(i + 1) * _G]
            lo1 = jnp.minimum(t1, v)
            t1 = jnp.maximum(t1, v)
            lo2 = jnp.minimum(t2, lo1)
            t2 = jnp.maximum(t2, lo1)
            t3 = jnp.maximum(t3, lo2)
        for s in (4, 2, 1):
            b1 = pltpu.roll(t1, s, 0)
            b2 = pltpu.roll(t2, s, 0)
            b3 = pltpu.roll(t3, s, 0)
            t1, t2, t3 = _merge3(t1, t2, t3, b1, b2, b3)
        tsum = (t1 + t2 + t3)[0:1]
        per_q = tsum.reshape(QC, P).sum(axis=-1) * (1.0 / (P * _K))
        per_way.append(per_q)
    o_ref[0, 0] = jnp.stack(per_way, axis=-1)


def kernel(query_fea, support_fea):
    b, q, p, d = query_fea.shape
    _, way, shot, _, _ = support_fea.shape
    support = support_fea.reshape(b, way, shot * p, d)

    qc = q
    for cand in (30, 15, 10, 6, 5, 3, 2, 1):
        if q % cand == 0:
            qc = cand
            break

    out = pl.pallas_call(
        _fused_kernel,
        grid=(b, q // qc),
        in_specs=[
            pl.BlockSpec((1, qc, p, d), lambda i, j: (i, j, 0, 0)),
            pl.BlockSpec((1, way, shot * p, d), lambda i, j: (i, 0, 0, 0)),
        ],
        out_specs=pl.BlockSpec((1, 1, qc, way), lambda i, j: (i, j, 0, 0)),
        out_shape=jax.ShapeDtypeStruct((b, q // qc, qc, way), jnp.float32),
        compiler_params=pltpu.CompilerParams(
            dimension_semantics=("parallel", "parallel"),
        ),
    )(query_fea, support)
    return out.reshape(b, q, way)
